# baseline (device time: 39056 ns/iter reference)
import jax
import jax.numpy as jnp
from jax import lax
from jax.experimental import pallas as pl
from jax.experimental.pallas import tpu as pltpu

N_DEV = 4
B_LOC = 2
H_TOT = 16
H_LOC = 4
H_HALF = 2
SQ = 128
DH = 64
DHH = H_HALF * DH
SCALE = 0.125


def kernel(x, Wq, K_ext, V_ext, Wo):
    def body(x_ref, wq_ref, k_any, v_any, wo_ref, out_ref,
             snd_wq_cw, snd_wq_ccw,
             l_wq_cw, l_wo_cw, l_wq_ccw, l_wo_ccw,
             r_wq_cw, r_wo_cw, r_wq_ccw, r_wo_ccw,
             o_wq_cw, o_wo_cw, o_wq_ccw, o_wo_ccw,
             kt, vt, kv_sems, send_sems, recv_sems):
        me = lax.axis_index("i")
        right = lax.rem(me + 1, N_DEV)
        left = lax.rem(me + N_DEV - 1, N_DEV)
        opp = lax.rem(me + 2, N_DEV)

        kv_dmas = []
        for src_any, dst, kv in ((k_any, kt, 0), (v_any, vt, 1)):
            for b in range(B_LOC):
                for ih in range(H_TOT):
                    d = pltpu.make_async_copy(
                        src_any.at[pl.ds(me * B_LOC + b, 1), :, ih, :],
                        dst.at[b, ih],
                        kv_sems.at[kv, b, ih],
                    )
                    d.start()
                    kv_dmas.append(d)

        barrier_sem = pltpu.get_barrier_semaphore()
        for nbr in (left, right):
            pl.semaphore_signal(
                barrier_sem, inc=1,
                device_id=(nbr,), device_id_type=pl.DeviceIdType.MESH,
            )
        pl.semaphore_wait(barrier_sem, 2)

        snd_wq_cw[...] = wq_ref[:, :DHH]
        snd_wq_ccw[...] = wq_ref[:, DHH:]

        def mk(src, dst, idx, nbr):
            return pltpu.make_async_remote_copy(
                src_ref=src, dst_ref=dst,
                send_sem=send_sems.at[idx], recv_sem=recv_sems.at[idx],
                device_id=(nbr,), device_id_type=pl.DeviceIdType.MESH,
            )

        wo_cw_src = wo_ref.at[pl.ds(0, DHH)]
        wo_ccw_src = wo_ref.at[pl.ds(DHH, DHH)]

        hop1 = [
            mk(snd_wq_cw, l_wq_cw, 0, right),
            mk(wo_cw_src, l_wo_cw, 1, right),
            mk(snd_wq_ccw, l_wq_ccw, 2, right),
            mk(wo_ccw_src, l_wo_ccw, 3, right),
            mk(snd_wq_ccw, r_wq_ccw, 4, left),
            mk(wo_ccw_src, r_wo_ccw, 5, left),
            mk(snd_wq_cw, r_wq_cw, 6, left),
            mk(wo_cw_src, r_wo_cw, 7, left),
        ]
        for r in hop1:
            r.start()

        for d in kv_dmas:
            d.wait()

        def attend_half(b, q_half, o, head_off, wo_half):
            ctx_parts = []
            for hh in range(H_HALF):
                ih = o * H_LOC + head_off + hh
                k = kt[b, pl.ds(ih, 1)].reshape(SQ, DH)
                v = vt[b, pl.ds(ih, 1)].reshape(SQ, DH)
                q = q_half[:, hh * DH:(hh + 1) * DH]
                s = lax.dot_general(
                    q, k, (((1,), (1,)), ((), ())),
                    preferred_element_type=jnp.float32,
                ) * SCALE
                m = jnp.max(s, axis=-1, keepdims=True)
                w = jnp.exp(s - m)
                w = w / jnp.sum(w, axis=-1, keepdims=True)
                ctx_parts.append(
                    jnp.dot(w, v, preferred_element_type=jnp.float32)
                )
            ctx = jnp.concatenate(ctx_parts, axis=1)
            return jnp.dot(ctx, wo_half, preferred_element_type=jnp.float32)

        def compute_half(origin, head_off, wq_half_ref, wo_half_ref,
                         init=False):
            wq_half = wq_half_ref[...]
            wo_half = wo_half_ref[...]
            for b in range(B_LOC):
                q = jnp.dot(
                    x_ref[b], wq_half, preferred_element_type=jnp.float32
                )
                c = attend_half(b, q, origin, head_off, wo_half)
                if init:
                    out_ref[b] = c
                else:
                    out_ref[b] = out_ref[b] + c

        compute_half(me, 0, snd_wq_cw, wo_cw_src, init=True)
        compute_half(me, H_HALF, snd_wq_ccw, wo_ccw_src)

        hop1[0].wait_recv()
        hop1[1].wait_recv()
        hop2 = [
            mk(l_wq_cw, o_wq_cw, 8, right),
            mk(l_wo_cw, o_wo_cw, 9, right),
        ]
        hop1[4].wait_recv()
        hop1[5].wait_recv()
        hop2 += [
            mk(r_wq_ccw, o_wq_ccw, 10, left),
            mk(r_wo_ccw, o_wo_ccw, 11, left),
        ]
        for r in hop2:
            r.start()

        compute_half(left, 0, l_wq_cw, l_wo_cw)
        compute_half(right, H_HALF, r_wq_ccw, r_wo_ccw)

        hop1[2].wait_recv()
        hop1[3].wait_recv()
        compute_half(left, H_HALF, l_wq_ccw, l_wo_ccw)

        hop1[6].wait_recv()
        hop1[7].wait_recv()
        compute_half(right, 0, r_wq_cw, r_wo_cw)

        hop2[0].wait_recv()
        hop2[1].wait_recv()
        compute_half(opp, 0, o_wq_cw, o_wo_cw)

        hop2[2].wait_recv()
        hop2[3].wait_recv()
        compute_half(opp, H_HALF, o_wq_ccw, o_wo_ccw)

        for r in hop1 + hop2:
            r.wait_send()

    wq_half_t = pltpu.VMEM((Wq.shape[0], DHH), jnp.float32)
    wo_half_t = pltpu.VMEM((DHH, Wo.shape[1]), jnp.float32)
    kv_t = pltpu.VMEM((B_LOC, H_TOT, 1, SQ, DH), jnp.float32)
    return pl.pallas_call(
        body,
        out_shape=jax.ShapeDtypeStruct(x.shape, jnp.float32),
        in_specs=[
            pl.BlockSpec(memory_space=pltpu.VMEM),
            pl.BlockSpec(memory_space=pltpu.VMEM),
            pl.BlockSpec(memory_space=pl.ANY),
            pl.BlockSpec(memory_space=pl.ANY),
            pl.BlockSpec(memory_space=pltpu.VMEM),
        ],
        out_specs=pl.BlockSpec(memory_space=pltpu.VMEM),
        scratch_shapes=[
            wq_half_t, wq_half_t,
            wq_half_t, wo_half_t, wq_half_t, wo_half_t,
            wq_half_t, wo_half_t, wq_half_t, wo_half_t,
            wq_half_t, wo_half_t, wq_half_t, wo_half_t,
            kv_t, kv_t,
            pltpu.SemaphoreType.DMA((2, B_LOC, H_TOT)),
            pltpu.SemaphoreType.DMA((12,)),
            pltpu.SemaphoreType.DMA((12,)),
        ],
        compiler_params=pltpu.CompilerParams(collective_id=0),
    )(x, Wq, K_ext, V_ext, Wo)


# device time: 36417 ns/iter; 1.0725x vs baseline; 1.0725x over previous
import jax
import jax.numpy as jnp
from jax import lax
from jax.experimental import pallas as pl
from jax.experimental.pallas import tpu as pltpu

N_DEV = 4
B_LOC = 2
H_TOT = 16
H_LOC = 4
H_HALF = 2
SQ = 128
DH = 64
DHH = H_HALF * DH
SCALE = 0.125


def kernel(x, Wq, K_ext, V_ext, Wo):
    def body(x_ref, wq_ref, k_any, v_any, wo_ref, out_ref,
             snd_wq_cw, snd_wq_ccw,
             l_wq_cw, l_wo_cw, l_wq_ccw, l_wo_ccw,
             r_wq_cw, r_wo_cw, r_wq_ccw, r_wo_ccw,
             o_wq_cw, o_wo_cw, o_wq_ccw, o_wo_ccw,
             k_raw, v_raw, kt, vt, raw_sems, kv_sems, send_sems, recv_sems):
        me = lax.axis_index("i")
        right = lax.rem(me + 1, N_DEV)
        left = lax.rem(me + N_DEV - 1, N_DEV)
        opp = lax.rem(me + 2, N_DEV)

        raw_dmas = [
            pltpu.make_async_copy(
                k_any.at[pl.ds(me * B_LOC, B_LOC)], k_raw, raw_sems.at[0]
            ),
            pltpu.make_async_copy(
                v_any.at[pl.ds(me * B_LOC, B_LOC)], v_raw, raw_sems.at[1]
            ),
        ]
        for d in raw_dmas:
            d.start()

        barrier_sem = pltpu.get_barrier_semaphore()
        for nbr in (left, right):
            pl.semaphore_signal(
                barrier_sem, inc=1,
                device_id=(nbr,), device_id_type=pl.DeviceIdType.MESH,
            )
        pl.semaphore_wait(barrier_sem, 2)

        snd_wq_cw[...] = wq_ref[:, :DHH]
        snd_wq_ccw[...] = wq_ref[:, DHH:]

        def mk(src, dst, idx, nbr):
            return pltpu.make_async_remote_copy(
                src_ref=src, dst_ref=dst,
                send_sem=send_sems.at[idx], recv_sem=recv_sems.at[idx],
                device_id=(nbr,), device_id_type=pl.DeviceIdType.MESH,
            )

        wo_cw_src = wo_ref.at[pl.ds(0, DHH)]
        wo_ccw_src = wo_ref.at[pl.ds(DHH, DHH)]

        hop1 = [
            mk(snd_wq_cw, l_wq_cw, 0, right),
            mk(wo_cw_src, l_wo_cw, 1, right),
            mk(snd_wq_ccw, l_wq_ccw, 2, right),
            mk(wo_ccw_src, l_wo_ccw, 3, right),
            mk(snd_wq_ccw, r_wq_ccw, 4, left),
            mk(wo_ccw_src, r_wo_ccw, 5, left),
            mk(snd_wq_cw, r_wq_cw, 6, left),
            mk(wo_cw_src, r_wo_cw, 7, left),
        ]
        for r in hop1:
            r.start()

        for d in raw_dmas:
            d.wait()
        kv_dmas = []
        for src, dst, kv in ((k_raw, kt, 0), (v_raw, vt, 1)):
            for b in range(B_LOC):
                for ih in range(H_TOT):
                    d = pltpu.make_async_copy(
                        src.at[b, :, ih, :],
                        dst.at[b, ih],
                        kv_sems.at[kv, b, ih],
                    )
                    d.start()
                    kv_dmas.append(d)
        for d in kv_dmas:
            d.wait()

        def attend_half(b, q_half, o, head_off, wo_half):
            ctx_parts = []
            for hh in range(H_HALF):
                ih = o * H_LOC + head_off + hh
                k = kt[b, pl.ds(ih, 1)].reshape(SQ, DH)
                v = vt[b, pl.ds(ih, 1)].reshape(SQ, DH)
                q = q_half[:, hh * DH:(hh + 1) * DH]
                s = lax.dot_general(
                    q, k, (((1,), (1,)), ((), ())),
                    preferred_element_type=jnp.float32,
                ) * SCALE
                m = jnp.max(s, axis=-1, keepdims=True)
                w = jnp.exp(s - m)
                w = w / jnp.sum(w, axis=-1, keepdims=True)
                ctx_parts.append(
                    jnp.dot(w, v, preferred_element_type=jnp.float32)
                )
            ctx = jnp.concatenate(ctx_parts, axis=1)
            return jnp.dot(ctx, wo_half, preferred_element_type=jnp.float32)

        def compute_half(origin, head_off, wq_half_ref, wo_half_ref,
                         init=False):
            wq_half = wq_half_ref[...]
            wo_half = wo_half_ref[...]
            for b in range(B_LOC):
                q = jnp.dot(
                    x_ref[b], wq_half, preferred_element_type=jnp.float32
                )
                c = attend_half(b, q, origin, head_off, wo_half)
                if init:
                    out_ref[b] = c
                else:
                    out_ref[b] = out_ref[b] + c

        compute_half(me, 0, snd_wq_cw, wo_cw_src, init=True)
        compute_half(me, H_HALF, snd_wq_ccw, wo_ccw_src)

        hop1[0].wait_recv()
        hop1[1].wait_recv()
        hop2 = [
            mk(l_wq_cw, o_wq_cw, 8, right),
            mk(l_wo_cw, o_wo_cw, 9, right),
        ]
        hop1[4].wait_recv()
        hop1[5].wait_recv()
        hop2 += [
            mk(r_wq_ccw, o_wq_ccw, 10, left),
            mk(r_wo_ccw, o_wo_ccw, 11, left),
        ]
        for r in hop2:
            r.start()

        compute_half(left, 0, l_wq_cw, l_wo_cw)
        compute_half(right, H_HALF, r_wq_ccw, r_wo_ccw)

        hop1[2].wait_recv()
        hop1[3].wait_recv()
        compute_half(left, H_HALF, l_wq_ccw, l_wo_ccw)

        hop1[6].wait_recv()
        hop1[7].wait_recv()
        compute_half(right, 0, r_wq_cw, r_wo_cw)

        hop2[0].wait_recv()
        hop2[1].wait_recv()
        compute_half(opp, 0, o_wq_cw, o_wo_cw)

        hop2[2].wait_recv()
        hop2[3].wait_recv()
        compute_half(opp, H_HALF, o_wq_ccw, o_wo_ccw)

        for r in hop1 + hop2:
            r.wait_send()

    wq_half_t = pltpu.VMEM((Wq.shape[0], DHH), jnp.float32)
    wo_half_t = pltpu.VMEM((DHH, Wo.shape[1]), jnp.float32)
    kv_raw_t = pltpu.VMEM((B_LOC, SQ, H_TOT, DH), jnp.float32)
    kv_t = pltpu.VMEM((B_LOC, H_TOT, SQ, DH), jnp.float32)
    return pl.pallas_call(
        body,
        out_shape=jax.ShapeDtypeStruct(x.shape, jnp.float32),
        in_specs=[
            pl.BlockSpec(memory_space=pltpu.VMEM),
            pl.BlockSpec(memory_space=pltpu.VMEM),
            pl.BlockSpec(memory_space=pl.ANY),
            pl.BlockSpec(memory_space=pl.ANY),
            pl.BlockSpec(memory_space=pltpu.VMEM),
        ],
        out_specs=pl.BlockSpec(memory_space=pltpu.VMEM),
        scratch_shapes=[
            wq_half_t, wq_half_t,
            wq_half_t, wo_half_t, wq_half_t, wo_half_t,
            wq_half_t, wo_half_t, wq_half_t, wo_half_t,
            wq_half_t, wo_half_t, wq_half_t, wo_half_t,
            kv_raw_t, kv_raw_t,
            kv_t, kv_t,
            pltpu.SemaphoreType.DMA((2,)),
            pltpu.SemaphoreType.DMA((2, B_LOC, H_TOT)),
            pltpu.SemaphoreType.DMA((12,)),
            pltpu.SemaphoreType.DMA((12,)),
        ],
        compiler_params=pltpu.CompilerParams(collective_id=0),
    )(x, Wq, K_ext, V_ext, Wo)


# device time: 35955 ns/iter; 1.0862x vs baseline; 1.0128x over previous
import jax
import jax.numpy as jnp
from jax import lax
from jax.experimental import pallas as pl
from jax.experimental.pallas import tpu as pltpu

N_DEV = 4
B_LOC = 2
H_TOT = 16
H_LOC = 4
H_HALF = 2
SQ = 128
DH = 64
DHH = H_HALF * DH
SCALE = 0.125


def kernel(x, Wq, K_ext, V_ext, Wo):
    def body(x_ref, wq_ref, k_any, v_any, wo_ref, out_ref,
             snd_wq_cw, snd_wq_ccw,
             l_wq_cw, l_wo_cw, l_wq_ccw, l_wo_ccw,
             r_wq_cw, r_wo_cw, r_wq_ccw, r_wo_ccw,
             o_wq_cw, o_wo_cw, o_wq_ccw, o_wo_ccw,
             k_raw, v_raw, raw_sems, send_sems, recv_sems):
        me = lax.axis_index("i")
        right = lax.rem(me + 1, N_DEV)
        left = lax.rem(me + N_DEV - 1, N_DEV)
        opp = lax.rem(me + 2, N_DEV)

        raw_dmas = [
            pltpu.make_async_copy(
                k_any.at[pl.ds(me * B_LOC, B_LOC)], k_raw, raw_sems.at[0]
            ),
            pltpu.make_async_copy(
                v_any.at[pl.ds(me * B_LOC, B_LOC)], v_raw, raw_sems.at[1]
            ),
        ]
        for d in raw_dmas:
            d.start()

        barrier_sem = pltpu.get_barrier_semaphore()
        for nbr in (left, right):
            pl.semaphore_signal(
                barrier_sem, inc=1,
                device_id=(nbr,), device_id_type=pl.DeviceIdType.MESH,
            )
        pl.semaphore_wait(barrier_sem, 2)

        snd_wq_cw[...] = wq_ref[:, :DHH]
        snd_wq_ccw[...] = wq_ref[:, DHH:]

        def mk(src, dst, idx, nbr):
            return pltpu.make_async_remote_copy(
                src_ref=src, dst_ref=dst,
                send_sem=send_sems.at[idx], recv_sem=recv_sems.at[idx],
                device_id=(nbr,), device_id_type=pl.DeviceIdType.MESH,
            )

        wo_cw_src = wo_ref.at[pl.ds(0, DHH)]
        wo_ccw_src = wo_ref.at[pl.ds(DHH, DHH)]

        hop1 = [
            mk(snd_wq_cw, l_wq_cw, 0, right),
            mk(wo_cw_src, l_wo_cw, 1, right),
            mk(snd_wq_ccw, l_wq_ccw, 2, right),
            mk(wo_ccw_src, l_wo_ccw, 3, right),
            mk(snd_wq_ccw, r_wq_ccw, 4, left),
            mk(wo_ccw_src, r_wo_ccw, 5, left),
            mk(snd_wq_cw, r_wq_cw, 6, left),
            mk(wo_cw_src, r_wo_cw, 7, left),
        ]
        for r in hop1:
            r.start()

        for d in raw_dmas:
            d.wait()

        def attend_half(b, q_half, o, head_off, wo_half):
            ctx_parts = []
            for hh in range(H_HALF):
                ih = o * H_LOC + head_off + hh
                k = k_raw[b, :, pl.ds(ih, 1), :].reshape(SQ, DH)
                v = v_raw[b, :, pl.ds(ih, 1), :].reshape(SQ, DH)
                q = q_half[:, hh * DH:(hh + 1) * DH]
                s = lax.dot_general(
                    q, k, (((1,), (1,)), ((), ())),
                    preferred_element_type=jnp.float32,
                ) * SCALE
                m = jnp.max(s, axis=-1, keepdims=True)
                w = jnp.exp(s - m)
                w = w / jnp.sum(w, axis=-1, keepdims=True)
                ctx_parts.append(
                    jnp.dot(w, v, preferred_element_type=jnp.float32)
                )
            ctx = jnp.concatenate(ctx_parts, axis=1)
            return jnp.dot(ctx, wo_half, preferred_element_type=jnp.float32)

        def compute_half(origin, head_off, wq_half_ref, wo_half_ref,
                         init=False):
            wq_half = wq_half_ref[...]
            wo_half = wo_half_ref[...]
            for b in range(B_LOC):
                q = jnp.dot(
                    x_ref[b], wq_half, preferred_element_type=jnp.float32
                )
                c = attend_half(b, q, origin, head_off, wo_half)
                if init:
                    out_ref[b] = c
                else:
                    out_ref[b] = out_ref[b] + c

        compute_half(me, 0, snd_wq_cw, wo_cw_src, init=True)
        compute_half(me, H_HALF, snd_wq_ccw, wo_ccw_src)

        hop1[0].wait_recv()
        hop1[1].wait_recv()
        hop2 = [
            mk(l_wq_cw, o_wq_cw, 8, right),
            mk(l_wo_cw, o_wo_cw, 9, right),
        ]
        hop1[4].wait_recv()
        hop1[5].wait_recv()
        hop2 += [
            mk(r_wq_ccw, o_wq_ccw, 10, left),
            mk(r_wo_ccw, o_wo_ccw, 11, left),
        ]
        for r in hop2:
            r.start()

        compute_half(left, 0, l_wq_cw, l_wo_cw)
        compute_half(right, H_HALF, r_wq_ccw, r_wo_ccw)

        hop1[2].wait_recv()
        hop1[3].wait_recv()
        compute_half(left, H_HALF, l_wq_ccw, l_wo_ccw)

        hop1[6].wait_recv()
        hop1[7].wait_recv()
        compute_half(right, 0, r_wq_cw, r_wo_cw)

        hop2[0].wait_recv()
        hop2[1].wait_recv()
        compute_half(opp, 0, o_wq_cw, o_wo_cw)

        hop2[2].wait_recv()
        hop2[3].wait_recv()
        compute_half(opp, H_HALF, o_wq_ccw, o_wo_ccw)

        for r in hop1 + hop2:
            r.wait_send()

    wq_half_t = pltpu.VMEM((Wq.shape[0], DHH), jnp.float32)
    wo_half_t = pltpu.VMEM((DHH, Wo.shape[1]), jnp.float32)
    kv_raw_t = pltpu.VMEM((B_LOC, SQ, H_TOT, DH), jnp.float32)
    return pl.pallas_call(
        body,
        out_shape=jax.ShapeDtypeStruct(x.shape, jnp.float32),
        in_specs=[
            pl.BlockSpec(memory_space=pltpu.VMEM),
            pl.BlockSpec(memory_space=pltpu.VMEM),
            pl.BlockSpec(memory_space=pl.ANY),
            pl.BlockSpec(memory_space=pl.ANY),
            pl.BlockSpec(memory_space=pltpu.VMEM),
        ],
        out_specs=pl.BlockSpec(memory_space=pltpu.VMEM),
        scratch_shapes=[
            wq_half_t, wq_half_t,
            wq_half_t, wo_half_t, wq_half_t, wo_half_t,
            wq_half_t, wo_half_t, wq_half_t, wo_half_t,
            wq_half_t, wo_half_t, wq_half_t, wo_half_t,
            kv_raw_t, kv_raw_t,
            pltpu.SemaphoreType.DMA((2,)),
            pltpu.SemaphoreType.DMA((12,)),
            pltpu.SemaphoreType.DMA((12,)),
        ],
        compiler_params=pltpu.CompilerParams(collective_id=0),
    )(x, Wq, K_ext, V_ext, Wo)


# device time: 29380 ns/iter; 1.3293x vs baseline; 1.2238x over previous
import jax
import jax.numpy as jnp
from jax import lax
from jax.experimental import pallas as pl
from jax.experimental.pallas import tpu as pltpu

N_DEV = 4
B_LOC = 2
H_TOT = 16
H_LOC = 4
H_HALF = 2
SQ = 128
DH = 64
DHH = H_HALF * DH
SCALE = 0.125


def kernel(x, Wq, K_ext, V_ext, Wo):
    my = lax.axis_index("i")
    K_my = lax.dynamic_slice_in_dim(K_ext, my * B_LOC, B_LOC, axis=0)
    V_my = lax.dynamic_slice_in_dim(V_ext, my * B_LOC, B_LOC, axis=0)
    K_t = jnp.transpose(K_my, (0, 2, 1, 3))
    V_t = jnp.transpose(V_my, (0, 2, 1, 3))

    def body(x_ref, wq_ref, k_any, v_any, wo_ref, out_ref,
             snd_wq_cw, snd_wq_ccw,
             l_wq_cw, l_wo_cw, l_wq_ccw, l_wo_ccw,
             r_wq_cw, r_wo_cw, r_wq_ccw, r_wo_ccw,
             o_wq_cw, o_wo_cw, o_wq_ccw, o_wo_ccw,
             kt, vt, raw_sems, send_sems, recv_sems):
        me = lax.axis_index("i")
        right = lax.rem(me + 1, N_DEV)
        left = lax.rem(me + N_DEV - 1, N_DEV)
        opp = lax.rem(me + 2, N_DEV)

        raw_dmas = [
            pltpu.make_async_copy(k_any, kt, raw_sems.at[0]),
            pltpu.make_async_copy(v_any, vt, raw_sems.at[1]),
        ]
        for d in raw_dmas:
            d.start()

        barrier_sem = pltpu.get_barrier_semaphore()
        for nbr in (left, right):
            pl.semaphore_signal(
                barrier_sem, inc=1,
                device_id=(nbr,), device_id_type=pl.DeviceIdType.MESH,
            )
        pl.semaphore_wait(barrier_sem, 2)

        snd_wq_cw[...] = wq_ref[:, :DHH]
        snd_wq_ccw[...] = wq_ref[:, DHH:]

        def mk(src, dst, idx, nbr):
            return pltpu.make_async_remote_copy(
                src_ref=src, dst_ref=dst,
                send_sem=send_sems.at[idx], recv_sem=recv_sems.at[idx],
                device_id=(nbr,), device_id_type=pl.DeviceIdType.MESH,
            )

        wo_cw_src = wo_ref.at[pl.ds(0, DHH)]
        wo_ccw_src = wo_ref.at[pl.ds(DHH, DHH)]

        hop1 = [
            mk(snd_wq_cw, l_wq_cw, 0, right),
            mk(wo_cw_src, l_wo_cw, 1, right),
            mk(snd_wq_ccw, l_wq_ccw, 2, right),
            mk(wo_ccw_src, l_wo_ccw, 3, right),
            mk(snd_wq_ccw, r_wq_ccw, 4, left),
            mk(wo_ccw_src, r_wo_ccw, 5, left),
            mk(snd_wq_cw, r_wq_cw, 6, left),
            mk(wo_cw_src, r_wo_cw, 7, left),
        ]
        for r in hop1:
            r.start()

        for d in raw_dmas:
            d.wait()

        def attend_half(b, q_half, o, head_off, wo_half):
            ctx_parts = []
            for hh in range(H_HALF):
                ih = o * H_LOC + head_off + hh
                k = kt[b, pl.ds(ih, 1)].reshape(SQ, DH)
                v = vt[b, pl.ds(ih, 1)].reshape(SQ, DH)
                q = q_half[:, hh * DH:(hh + 1) * DH]
                s = lax.dot_general(
                    q, k, (((1,), (1,)), ((), ())),
                    preferred_element_type=jnp.float32,
                ) * SCALE
                m = jnp.max(s, axis=-1, keepdims=True)
                w = jnp.exp(s - m)
                w = w / jnp.sum(w, axis=-1, keepdims=True)
                ctx_parts.append(
                    jnp.dot(w, v, preferred_element_type=jnp.float32)
                )
            ctx = jnp.concatenate(ctx_parts, axis=1)
            return jnp.dot(ctx, wo_half, preferred_element_type=jnp.float32)

        def compute_half(origin, head_off, wq_half_ref, wo_half_ref,
                         init=False):
            wq_half = wq_half_ref[...]
            wo_half = wo_half_ref[...]
            for b in range(B_LOC):
                q = jnp.dot(
                    x_ref[b], wq_half, preferred_element_type=jnp.float32
                )
                c = attend_half(b, q, origin, head_off, wo_half)
                if init:
                    out_ref[b] = c
                else:
                    out_ref[b] = out_ref[b] + c

        compute_half(me, 0, snd_wq_cw, wo_cw_src, init=True)
        compute_half(me, H_HALF, snd_wq_ccw, wo_ccw_src)

        hop1[0].wait_recv()
        hop1[1].wait_recv()
        hop2 = [
            mk(l_wq_cw, o_wq_cw, 8, right),
            mk(l_wo_cw, o_wo_cw, 9, right),
        ]
        hop1[4].wait_recv()
        hop1[5].wait_recv()
        hop2 += [
            mk(r_wq_ccw, o_wq_ccw, 10, left),
            mk(r_wo_ccw, o_wo_ccw, 11, left),
        ]
        for r in hop2:
            r.start()

        compute_half(left, 0, l_wq_cw, l_wo_cw)
        compute_half(right, H_HALF, r_wq_ccw, r_wo_ccw)

        hop1[2].wait_recv()
        hop1[3].wait_recv()
        compute_half(left, H_HALF, l_wq_ccw, l_wo_ccw)

        hop1[6].wait_recv()
        hop1[7].wait_recv()
        compute_half(right, 0, r_wq_cw, r_wo_cw)

        hop2[0].wait_recv()
        hop2[1].wait_recv()
        compute_half(opp, 0, o_wq_cw, o_wo_cw)

        hop2[2].wait_recv()
        hop2[3].wait_recv()
        compute_half(opp, H_HALF, o_wq_ccw, o_wo_ccw)

        for r in hop1 + hop2:
            r.wait_send()

    wq_half_t = pltpu.VMEM((Wq.shape[0], DHH), jnp.float32)
    wo_half_t = pltpu.VMEM((DHH, Wo.shape[1]), jnp.float32)
    kv_t = pltpu.VMEM((B_LOC, H_TOT, SQ, DH), jnp.float32)
    return pl.pallas_call(
        body,
        out_shape=jax.ShapeDtypeStruct(x.shape, jnp.float32),
        in_specs=[
            pl.BlockSpec(memory_space=pltpu.VMEM),
            pl.BlockSpec(memory_space=pltpu.VMEM),
            pl.BlockSpec(memory_space=pl.ANY),
            pl.BlockSpec(memory_space=pl.ANY),
            pl.BlockSpec(memory_space=pltpu.VMEM),
        ],
        out_specs=pl.BlockSpec(memory_space=pltpu.VMEM),
        scratch_shapes=[
            wq_half_t, wq_half_t,
            wq_half_t, wo_half_t, wq_half_t, wo_half_t,
            wq_half_t, wo_half_t, wq_half_t, wo_half_t,
            wq_half_t, wo_half_t, wq_half_t, wo_half_t,
            kv_t, kv_t,
            pltpu.SemaphoreType.DMA((2,)),
            pltpu.SemaphoreType.DMA((12,)),
            pltpu.SemaphoreType.DMA((12,)),
        ],
        compiler_params=pltpu.CompilerParams(collective_id=0),
    )(x, Wq, K_t, V_t, Wo)


# device time: 24408 ns/iter; 1.6001x vs baseline; 1.2037x over previous
import jax
import jax.numpy as jnp
from jax import lax
from jax.experimental import pallas as pl
from jax.experimental.pallas import tpu as pltpu

N_DEV = 4
B_LOC = 2
H_TOT = 16
H_LOC = 4
H_HALF = 2
SQ = 128
DH = 64
DHH = H_HALF * DH
SCALE = 0.125


def kernel(x, Wq, K_ext, V_ext, Wo):
    my = lax.axis_index("i")
    x16 = x.astype(jnp.bfloat16)
    Wq16 = Wq.astype(jnp.bfloat16)
    Wo16 = Wo.astype(jnp.bfloat16)
    K_my = lax.dynamic_slice_in_dim(K_ext, my * B_LOC, B_LOC, axis=0)
    V_my = lax.dynamic_slice_in_dim(V_ext, my * B_LOC, B_LOC, axis=0)
    K_t = jnp.transpose(K_my.astype(jnp.bfloat16), (0, 2, 1, 3))
    V_t = jnp.transpose(V_my.astype(jnp.bfloat16), (0, 2, 1, 3))

    def body(x_ref, wq_ref, k_any, v_any, wo_ref, out_ref,
             snd_wq_cw, snd_wq_ccw,
             l_wq_cw, l_wo_cw, l_wq_ccw, l_wo_ccw,
             r_wq_cw, r_wo_cw, r_wq_ccw, r_wo_ccw,
             o_wq_cw, o_wo_cw, o_wq_ccw, o_wo_ccw,
             kt, vt, raw_sems, send_sems, recv_sems):
        me = lax.axis_index("i")
        right = lax.rem(me + 1, N_DEV)
        left = lax.rem(me + N_DEV - 1, N_DEV)
        opp = lax.rem(me + 2, N_DEV)

        raw_dmas = [
            pltpu.make_async_copy(k_any, kt, raw_sems.at[0]),
            pltpu.make_async_copy(v_any, vt, raw_sems.at[1]),
        ]
        for d in raw_dmas:
            d.start()

        barrier_sem = pltpu.get_barrier_semaphore()
        for nbr in (left, right):
            pl.semaphore_signal(
                barrier_sem, inc=1,
                device_id=(nbr,), device_id_type=pl.DeviceIdType.MESH,
            )
        pl.semaphore_wait(barrier_sem, 2)

        snd_wq_cw[...] = wq_ref[:, :DHH]
        snd_wq_ccw[...] = wq_ref[:, DHH:]

        def mk(src, dst, idx, nbr):
            return pltpu.make_async_remote_copy(
                src_ref=src, dst_ref=dst,
                send_sem=send_sems.at[idx], recv_sem=recv_sems.at[idx],
                device_id=(nbr,), device_id_type=pl.DeviceIdType.MESH,
            )

        wo_cw_src = wo_ref.at[pl.ds(0, DHH)]
        wo_ccw_src = wo_ref.at[pl.ds(DHH, DHH)]

        hop1 = [
            mk(snd_wq_cw, l_wq_cw, 0, right),
            mk(wo_cw_src, l_wo_cw, 1, right),
            mk(snd_wq_ccw, l_wq_ccw, 2, right),
            mk(wo_ccw_src, l_wo_ccw, 3, right),
            mk(snd_wq_ccw, r_wq_ccw, 4, left),
            mk(wo_ccw_src, r_wo_ccw, 5, left),
            mk(snd_wq_cw, r_wq_cw, 6, left),
            mk(wo_cw_src, r_wo_cw, 7, left),
        ]
        for r in hop1:
            r.start()

        for d in raw_dmas:
            d.wait()

        def attend_half(b, q_half, o, head_off, wo_half):
            ctx_parts = []
            for hh in range(H_HALF):
                ih = o * H_LOC + head_off + hh
                k = kt[b, pl.ds(ih, 1)].reshape(SQ, DH)
                v = vt[b, pl.ds(ih, 1)].reshape(SQ, DH)
                q = q_half[:, hh * DH:(hh + 1) * DH]
                s = lax.dot_general(
                    q, k, (((1,), (1,)), ((), ())),
                    preferred_element_type=jnp.float32,
                ) * SCALE
                m = jnp.max(s, axis=-1, keepdims=True)
                w = jnp.exp(s - m)
                w = (w / jnp.sum(w, axis=-1, keepdims=True)).astype(
                    jnp.bfloat16
                )
                ctx_parts.append(
                    jnp.dot(w, v, preferred_element_type=jnp.float32)
                )
            ctx = jnp.concatenate(ctx_parts, axis=1).astype(jnp.bfloat16)
            return jnp.dot(ctx, wo_half, preferred_element_type=jnp.float32)

        def compute_half(origin, head_off, wq_half_ref, wo_half_ref,
                         init=False):
            wq_half = wq_half_ref[...]
            wo_half = wo_half_ref[...]
            for b in range(B_LOC):
                q = jnp.dot(
                    x_ref[b], wq_half, preferred_element_type=jnp.float32
                ).astype(jnp.bfloat16)
                c = attend_half(b, q, origin, head_off, wo_half)
                if init:
                    out_ref[b] = c
                else:
                    out_ref[b] = out_ref[b] + c

        compute_half(me, 0, snd_wq_cw, wo_cw_src, init=True)
        compute_half(me, H_HALF, snd_wq_ccw, wo_ccw_src)

        hop1[0].wait_recv()
        hop1[1].wait_recv()
        hop2 = [
            mk(l_wq_cw, o_wq_cw, 8, right),
            mk(l_wo_cw, o_wo_cw, 9, right),
        ]
        hop1[4].wait_recv()
        hop1[5].wait_recv()
        hop2 += [
            mk(r_wq_ccw, o_wq_ccw, 10, left),
            mk(r_wo_ccw, o_wo_ccw, 11, left),
        ]
        for r in hop2:
            r.start()

        compute_half(left, 0, l_wq_cw, l_wo_cw)
        compute_half(right, H_HALF, r_wq_ccw, r_wo_ccw)

        hop1[2].wait_recv()
        hop1[3].wait_recv()
        compute_half(left, H_HALF, l_wq_ccw, l_wo_ccw)

        hop1[6].wait_recv()
        hop1[7].wait_recv()
        compute_half(right, 0, r_wq_cw, r_wo_cw)

        hop2[0].wait_recv()
        hop2[1].wait_recv()
        compute_half(opp, 0, o_wq_cw, o_wo_cw)

        hop2[2].wait_recv()
        hop2[3].wait_recv()
        compute_half(opp, H_HALF, o_wq_ccw, o_wo_ccw)

        for r in hop1 + hop2:
            r.wait_send()

    wq_half_t = pltpu.VMEM((Wq.shape[0], DHH), jnp.bfloat16)
    wo_half_t = pltpu.VMEM((DHH, Wo.shape[1]), jnp.bfloat16)
    kv_t = pltpu.VMEM((B_LOC, H_TOT, SQ, DH), jnp.bfloat16)
    return pl.pallas_call(
        body,
        out_shape=jax.ShapeDtypeStruct(x.shape, jnp.float32),
        in_specs=[
            pl.BlockSpec(memory_space=pltpu.VMEM),
            pl.BlockSpec(memory_space=pltpu.VMEM),
            pl.BlockSpec(memory_space=pl.ANY),
            pl.BlockSpec(memory_space=pl.ANY),
            pl.BlockSpec(memory_space=pltpu.VMEM),
        ],
        out_specs=pl.BlockSpec(memory_space=pltpu.VMEM),
        scratch_shapes=[
            wq_half_t, wq_half_t,
            wq_half_t, wo_half_t, wq_half_t, wo_half_t,
            wq_half_t, wo_half_t, wq_half_t, wo_half_t,
            wq_half_t, wo_half_t, wq_half_t, wo_half_t,
            kv_t, kv_t,
            pltpu.SemaphoreType.DMA((2,)),
            pltpu.SemaphoreType.DMA((12,)),
            pltpu.SemaphoreType.DMA((12,)),
        ],
        compiler_params=pltpu.CompilerParams(collective_id=0),
    )(x16, Wq16, K_t, V_t, Wo16)


# device time: 21270 ns/iter; 1.8362x vs baseline; 1.1475x over previous
import jax
import jax.numpy as jnp
from jax import lax
from jax.experimental import pallas as pl
from jax.experimental.pallas import tpu as pltpu

N_DEV = 4
B_LOC = 2
H_TOT = 16
H_LOC = 4
H_HALF = 2
SQ = 128
DH = 64
DHH = H_HALF * DH
SCALE = 0.125


def kernel(x, Wq, K_ext, V_ext, Wo):
    my = lax.axis_index("i")
    K_my = lax.dynamic_slice_in_dim(K_ext, my * B_LOC, B_LOC, axis=0)
    V_my = lax.dynamic_slice_in_dim(V_ext, my * B_LOC, B_LOC, axis=0)
    K_t = jnp.transpose(K_my.astype(jnp.bfloat16), (0, 2, 1, 3))
    V_t = jnp.transpose(V_my.astype(jnp.bfloat16), (0, 2, 1, 3))

    def body(x_ref, wq_ref, k_any, v_any, wo_ref, out_ref,
             x16, snd_wq_cw, snd_wq_ccw, snd_wo_cw, snd_wo_ccw,
             l_wq_cw, l_wo_cw, l_wq_ccw, l_wo_ccw,
             r_wq_cw, r_wo_cw, r_wq_ccw, r_wo_ccw,
             o_wq_cw, o_wo_cw, o_wq_ccw, o_wo_ccw,
             kt, vt, raw_sems, send_sems, recv_sems):
        me = lax.axis_index("i")
        right = lax.rem(me + 1, N_DEV)
        left = lax.rem(me + N_DEV - 1, N_DEV)
        opp = lax.rem(me + 2, N_DEV)

        raw_dmas = [
            pltpu.make_async_copy(k_any, kt, raw_sems.at[0]),
            pltpu.make_async_copy(v_any, vt, raw_sems.at[1]),
        ]
        for d in raw_dmas:
            d.start()

        barrier_sem = pltpu.get_barrier_semaphore()
        for nbr in (left, right):
            pl.semaphore_signal(
                barrier_sem, inc=1,
                device_id=(nbr,), device_id_type=pl.DeviceIdType.MESH,
            )
        pl.semaphore_wait(barrier_sem, 2)

        snd_wq_cw[...] = wq_ref[:, :DHH].astype(jnp.bfloat16)
        snd_wq_ccw[...] = wq_ref[:, DHH:].astype(jnp.bfloat16)
        snd_wo_cw[...] = wo_ref[:DHH, :].astype(jnp.bfloat16)
        snd_wo_ccw[...] = wo_ref[DHH:, :].astype(jnp.bfloat16)

        def mk(src, dst, idx, nbr):
            return pltpu.make_async_remote_copy(
                src_ref=src, dst_ref=dst,
                send_sem=send_sems.at[idx], recv_sem=recv_sems.at[idx],
                device_id=(nbr,), device_id_type=pl.DeviceIdType.MESH,
            )

        hop1 = [
            mk(snd_wq_cw, l_wq_cw, 0, right),
            mk(snd_wo_cw, l_wo_cw, 1, right),
            mk(snd_wq_ccw, l_wq_ccw, 2, right),
            mk(snd_wo_ccw, l_wo_ccw, 3, right),
            mk(snd_wq_ccw, r_wq_ccw, 4, left),
            mk(snd_wo_ccw, r_wo_ccw, 5, left),
            mk(snd_wq_cw, r_wq_cw, 6, left),
            mk(snd_wo_cw, r_wo_cw, 7, left),
        ]
        for r in hop1:
            r.start()

        x16[...] = x_ref[...].astype(jnp.bfloat16)

        for d in raw_dmas:
            d.wait()

        def attend_half(b, q_half, o, head_off, wo_half):
            ctx_parts = []
            for hh in range(H_HALF):
                ih = o * H_LOC + head_off + hh
                k = kt[b, pl.ds(ih, 1)].reshape(SQ, DH)
                v = vt[b, pl.ds(ih, 1)].reshape(SQ, DH)
                q = q_half[:, hh * DH:(hh + 1) * DH]
                s = lax.dot_general(
                    q, k, (((1,), (1,)), ((), ())),
                    preferred_element_type=jnp.float32,
                ) * SCALE
                m = jnp.max(s, axis=-1, keepdims=True)
                w = jnp.exp(s - m)
                w = (w / jnp.sum(w, axis=-1, keepdims=True)).astype(
                    jnp.bfloat16
                )
                ctx_parts.append(
                    jnp.dot(w, v, preferred_element_type=jnp.float32)
                )
            ctx = jnp.concatenate(ctx_parts, axis=1).astype(jnp.bfloat16)
            return jnp.dot(ctx, wo_half, preferred_element_type=jnp.float32)

        def compute_half(origin, head_off, wq_half_ref, wo_half_ref,
                         init=False):
            wq_half = wq_half_ref[...]
            wo_half = wo_half_ref[...]
            for b in range(B_LOC):
                q = jnp.dot(
                    x16[b], wq_half, preferred_element_type=jnp.float32
                ).astype(jnp.bfloat16)
                c = attend_half(b, q, origin, head_off, wo_half)
                if init:
                    out_ref[b] = c
                else:
                    out_ref[b] = out_ref[b] + c

        compute_half(me, 0, snd_wq_cw, snd_wo_cw, init=True)
        compute_half(me, H_HALF, snd_wq_ccw, snd_wo_ccw)

        hop1[0].wait_recv()
        hop1[1].wait_recv()
        hop2 = [
            mk(l_wq_cw, o_wq_cw, 8, right),
            mk(l_wo_cw, o_wo_cw, 9, right),
        ]
        hop1[4].wait_recv()
        hop1[5].wait_recv()
        hop2 += [
            mk(r_wq_ccw, o_wq_ccw, 10, left),
            mk(r_wo_ccw, o_wo_ccw, 11, left),
        ]
        for r in hop2:
            r.start()

        compute_half(left, 0, l_wq_cw, l_wo_cw)
        compute_half(right, H_HALF, r_wq_ccw, r_wo_ccw)

        hop1[2].wait_recv()
        hop1[3].wait_recv()
        compute_half(left, H_HALF, l_wq_ccw, l_wo_ccw)

        hop1[6].wait_recv()
        hop1[7].wait_recv()
        compute_half(right, 0, r_wq_cw, r_wo_cw)

        hop2[0].wait_recv()
        hop2[1].wait_recv()
        compute_half(opp, 0, o_wq_cw, o_wo_cw)

        hop2[2].wait_recv()
        hop2[3].wait_recv()
        compute_half(opp, H_HALF, o_wq_ccw, o_wo_ccw)

        for r in hop1 + hop2:
            r.wait_send()

    wq_half_t = pltpu.VMEM((Wq.shape[0], DHH), jnp.bfloat16)
    wo_half_t = pltpu.VMEM((DHH, Wo.shape[1]), jnp.bfloat16)
    kv_t = pltpu.VMEM((B_LOC, H_TOT, SQ, DH), jnp.bfloat16)
    return pl.pallas_call(
        body,
        out_shape=jax.ShapeDtypeStruct(x.shape, jnp.float32),
        in_specs=[
            pl.BlockSpec(memory_space=pltpu.VMEM),
            pl.BlockSpec(memory_space=pltpu.VMEM),
            pl.BlockSpec(memory_space=pl.ANY),
            pl.BlockSpec(memory_space=pl.ANY),
            pl.BlockSpec(memory_space=pltpu.VMEM),
        ],
        out_specs=pl.BlockSpec(memory_space=pltpu.VMEM),
        scratch_shapes=[
            pltpu.VMEM(x.shape, jnp.bfloat16),
            wq_half_t, wq_half_t, wo_half_t, wo_half_t,
            wq_half_t, wo_half_t, wq_half_t, wo_half_t,
            wq_half_t, wo_half_t, wq_half_t, wo_half_t,
            wq_half_t, wo_half_t, wq_half_t, wo_half_t,
            kv_t, kv_t,
            pltpu.SemaphoreType.DMA((2,)),
            pltpu.SemaphoreType.DMA((12,)),
            pltpu.SemaphoreType.DMA((12,)),
        ],
        compiler_params=pltpu.CompilerParams(collective_id=0),
    )(x, Wq, K_t, V_t, Wo)
